# baseline (speedup 1.0000x reference)
import functools

import jax
import jax.numpy as jnp
from jax.experimental import pallas as pl


def _xw_body(x_ref, w_ref, h_ref):
    s = w_ref.shape[0]
    x = x_ref[...]
    for i in range(s):
        h_ref[i] = jnp.dot(
            x, w_ref[i], preferred_element_type=jnp.float32
        ).astype(jnp.bfloat16)


def _gcn_body(adj_ref, h_ref, b_ref, out_ref):
    s = adj_ref.shape[0]
    acc = jnp.dot(
        adj_ref[0].astype(jnp.bfloat16),
        h_ref[0],
        preferred_element_type=jnp.float32,
    )
    for i in range(1, s):
        acc = acc + jnp.dot(
            adj_ref[i].astype(jnp.bfloat16),
            h_ref[i],
            preferred_element_type=jnp.float32,
        )
    acc = acc + b_ref[...]
    norm = jnp.sqrt(jnp.sum(acc * acc, axis=1, keepdims=True))
    out_ref[...] = acc / jnp.maximum(norm, 1e-12)


@functools.partial(jax.jit, static_argnames=())
def kernel(features, adjs, W, b):
    n, d_in = features.shape
    s, _, d_out = W.shape

    bx = 2000
    h = pl.pallas_call(
        _xw_body,
        grid=(pl.cdiv(n, bx),),
        in_specs=[
            pl.BlockSpec((bx, d_in), lambda i: (i, 0)),
            pl.BlockSpec((s, d_in, d_out), lambda i: (0, 0, 0)),
        ],
        out_specs=pl.BlockSpec((s, bx, d_out), lambda i: (0, i, 0)),
        out_shape=jax.ShapeDtypeStruct((s, n, d_out), jnp.bfloat16),
    )(features, W)

    bi = 128
    out = pl.pallas_call(
        _gcn_body,
        grid=(pl.cdiv(n, bi),),
        in_specs=[
            pl.BlockSpec((s, bi, n), lambda i: (0, i, 0)),
            pl.BlockSpec((s, n, d_out), lambda i: (0, 0, 0)),
            pl.BlockSpec((1, d_out), lambda i: (0, 0)),
        ],
        out_specs=pl.BlockSpec((bi, d_out), lambda i: (i, 0)),
        out_shape=jax.ShapeDtypeStruct((n, d_out), jnp.float32),
    )(adjs, h, b.reshape(1, d_out))
    return out


# speedup vs baseline: 1.1127x; 1.1127x over previous
import functools

import jax
import jax.numpy as jnp
from jax.experimental import pallas as pl
from jax.experimental.pallas import tpu as pltpu


def _xw_body(x_ref, w_ref, h_ref):
    s = w_ref.shape[0]
    x = x_ref[...]
    for i in range(s):
        h_ref[i] = jnp.dot(
            x, w_ref[i], preferred_element_type=jnp.float32
        ).astype(jnp.bfloat16)


def _gcn_body(adj_ref, h_ref, b_ref, out_ref):
    s = adj_ref.shape[0]
    acc = jnp.dot(
        adj_ref[0].astype(jnp.bfloat16),
        h_ref[0],
        preferred_element_type=jnp.float32,
    )
    for i in range(1, s):
        acc = acc + jnp.dot(
            adj_ref[i].astype(jnp.bfloat16),
            h_ref[i],
            preferred_element_type=jnp.float32,
        )
    acc = acc + b_ref[...]
    norm = jnp.sqrt(jnp.sum(acc * acc, axis=1, keepdims=True))
    out_ref[...] = acc / jnp.maximum(norm, 1e-12)


@functools.partial(jax.jit, static_argnames=())
def kernel(features, adjs, W, b):
    n, d_in = features.shape
    s, _, d_out = W.shape

    bx = 2000
    h = pl.pallas_call(
        _xw_body,
        grid=(pl.cdiv(n, bx),),
        in_specs=[
            pl.BlockSpec((bx, d_in), lambda i: (i, 0)),
            pl.BlockSpec((s, d_in, d_out), lambda i: (0, 0, 0)),
        ],
        out_specs=pl.BlockSpec((s, bx, d_out), lambda i: (0, i, 0)),
        out_shape=jax.ShapeDtypeStruct((s, n, d_out), jnp.bfloat16),
    )(features, W)

    bi = 256
    out = pl.pallas_call(
        _gcn_body,
        grid=(pl.cdiv(n, bi),),
        in_specs=[
            pl.BlockSpec((s, bi, n), lambda i: (0, i, 0)),
            pl.BlockSpec((s, n, d_out), lambda i: (0, 0, 0)),
            pl.BlockSpec((1, d_out), lambda i: (0, 0)),
        ],
        out_specs=pl.BlockSpec((bi, d_out), lambda i: (i, 0)),
        out_shape=jax.ShapeDtypeStruct((n, d_out), jnp.float32),
        compiler_params=pltpu.CompilerParams(
            dimension_semantics=("parallel",),
        ),
    )(adjs, h, b.reshape(1, d_out))
    return out


# speedup vs baseline: 1.1645x; 1.0466x over previous
import functools

import jax
import jax.numpy as jnp
from jax.experimental import pallas as pl
from jax.experimental.pallas import tpu as pltpu


def _gcn_body(adj_ref, x_ref, w_ref, b_ref, out_ref, h_ref):
    s = adj_ref.shape[0]

    @pl.when(pl.program_id(0) == 0)
    def _compute_h():
        x = x_ref[...]
        for i in range(s):
            h_ref[i] = jnp.dot(
                x, w_ref[i], preferred_element_type=jnp.float32
            ).astype(jnp.bfloat16)

    acc = jnp.dot(
        adj_ref[0].astype(jnp.bfloat16),
        h_ref[0],
        preferred_element_type=jnp.float32,
    )
    for i in range(1, s):
        acc = acc + jnp.dot(
            adj_ref[i].astype(jnp.bfloat16),
            h_ref[i],
            preferred_element_type=jnp.float32,
        )
    acc = acc + b_ref[...]
    norm = jnp.sqrt(jnp.sum(acc * acc, axis=1, keepdims=True))
    out_ref[...] = acc / jnp.maximum(norm, 1e-12)


@functools.partial(jax.jit, static_argnames=())
def kernel(features, adjs, W, b):
    n, d_in = features.shape
    s, _, d_out = W.shape

    bi = 256
    out = pl.pallas_call(
        _gcn_body,
        grid=(pl.cdiv(n, bi),),
        in_specs=[
            pl.BlockSpec((s, bi, n), lambda i: (0, i, 0)),
            pl.BlockSpec((n, d_in), lambda i: (0, 0)),
            pl.BlockSpec((s, d_in, d_out), lambda i: (0, 0, 0)),
            pl.BlockSpec((1, d_out), lambda i: (0, 0)),
        ],
        out_specs=pl.BlockSpec((bi, d_out), lambda i: (i, 0)),
        out_shape=jax.ShapeDtypeStruct((n, d_out), jnp.float32),
        scratch_shapes=[pltpu.VMEM((s, n, d_out), jnp.bfloat16)],
        compiler_params=pltpu.CompilerParams(
            dimension_semantics=("arbitrary",),
        ),
    )(adjs, features, W, b.reshape(1, d_out))
    return out


# speedup vs baseline: 1.1751x; 1.0092x over previous
import functools

import jax
import jax.numpy as jnp
from jax.experimental import pallas as pl
from jax.experimental.pallas import tpu as pltpu


def _gcn_body(adj_ref, x_ref, w_ref, b_ref, out_ref, h_ref):
    s = adj_ref.shape[0]

    @pl.when(pl.program_id(0) == 0)
    def _compute_h():
        x = x_ref[...]
        for i in range(s):
            h_ref[i] = jnp.dot(
                x, w_ref[i], preferred_element_type=jnp.float32
            ).astype(jnp.bfloat16)

    acc = jnp.dot(
        adj_ref[0].astype(jnp.bfloat16),
        h_ref[0],
        preferred_element_type=jnp.float32,
    )
    for i in range(1, s):
        acc = acc + jnp.dot(
            adj_ref[i].astype(jnp.bfloat16),
            h_ref[i],
            preferred_element_type=jnp.float32,
        )
    acc = acc + b_ref[...]
    norm = jnp.sqrt(jnp.sum(acc * acc, axis=1, keepdims=True))
    out_ref[...] = acc / jnp.maximum(norm, 1e-12)


@functools.partial(jax.jit, static_argnames=())
def kernel(features, adjs, W, b):
    n, d_in = features.shape
    s, _, d_out = W.shape

    bi = 400
    out = pl.pallas_call(
        _gcn_body,
        grid=(pl.cdiv(n, bi),),
        in_specs=[
            pl.BlockSpec((s, bi, n), lambda i: (0, i, 0)),
            pl.BlockSpec((n, d_in), lambda i: (0, 0)),
            pl.BlockSpec((s, d_in, d_out), lambda i: (0, 0, 0)),
            pl.BlockSpec((1, d_out), lambda i: (0, 0)),
        ],
        out_specs=pl.BlockSpec((bi, d_out), lambda i: (i, 0)),
        out_shape=jax.ShapeDtypeStruct((n, d_out), jnp.float32),
        scratch_shapes=[pltpu.VMEM((s, n, d_out), jnp.bfloat16)],
        compiler_params=pltpu.CompilerParams(
            dimension_semantics=("arbitrary",),
        ),
    )(adjs, features, W, b.reshape(1, d_out))
    return out
